# baseline (speedup 1.0000x reference)
import jax
import jax.numpy as jnp
from jax.experimental import pallas as pl

N0, N1, EMB = 384, 384, 96
BI = 32


def _body(e0_ref, e1_ref, out_ref):
    out_ref[...] = e0_ref[...][:, None, :] + e1_ref[...][None, :, :]


def kernel(x, emb_0, emb_1):
    del x
    out = pl.pallas_call(
        _body,
        grid=(N0 // BI,),
        in_specs=[
            pl.BlockSpec((BI, EMB), lambda g: (g, 0)),
            pl.BlockSpec((N1, EMB), lambda g: (0, 0)),
        ],
        out_specs=pl.BlockSpec((BI, N1, EMB), lambda g: (g, 0, 0)),
        out_shape=jax.ShapeDtypeStruct((N0, N1, EMB), jnp.float32),
    )(emb_0, emb_1)
    return out[None]
